# baseline (speedup 1.0000x reference)
import functools

import jax
import jax.numpy as jnp
from jax import lax
from jax.experimental import pallas as pl
from jax.experimental.pallas import tpu as pltpu
from jax.experimental.pallas import tpu_sc as plsc

IN_W = 2048
OUT_W = 2048
P = 8
FANOUT = 4
G = 85
NPG = G * FANOUT
N_HEAD_G = 21
GPAD = 32
SEG = P * GPAD
HEAD_W = FANOUT * SEG
TOT_W = P * NPG

TB = 512
NW = 32
L = 16


def _gelu_exact(x):
    return 0.5 * x * (1.0 + jax.lax.erf(x * (2.0 ** -0.5)))


def _gelu_tanh(x):
    c = 0.7978845608028654
    return 0.5 * x * (1.0 + jnp.tanh(c * (x + 0.044715 * x * x * x)))


def _stage1_body(x_ref, w1_ref, w1h_ref, b1_ref, bh_ref, a_ref, dec_ref):
    x_bf = x_ref[...].astype(jnp.bfloat16)
    z = jax.lax.dot_general(
        x_bf, w1_ref[...], (((1,), (1,)), ((), ())),
        preferred_element_type=jnp.float32)
    a_ref[...] = _gelu_tanh(z + b1_ref[...]).astype(jnp.bfloat16)

    zh = jax.lax.dot_general(
        x_bf, w1h_ref[...], (((1,), (1,)), ((), ())),
        preferred_element_type=jnp.float32)
    ah = _gelu_exact(zh + bh_ref[...])
    a0 = ah[:, 0 * SEG:1 * SEG]
    a1 = ah[:, 1 * SEG:2 * SEG]
    a2 = ah[:, 2 * SEG:3 * SEG]
    a3 = ah[:, 3 * SEG:4 * SEG]
    one = jnp.float32(1.0)
    dec = jnp.where(a1 > a0, one, 0.0)
    m = jnp.maximum(a0, a1)
    dec = jnp.where(a2 > m, 2.0, dec)
    m = jnp.maximum(m, a2)
    dec_ref[...] = jnp.where(a3 > m, 3.0, dec)


@jax.jit
def _stage1(x, W1, W1h, b1, bh):
    B = x.shape[0]
    return pl.pallas_call(
        _stage1_body,
        grid=(B // TB,),
        in_specs=[
            pl.BlockSpec((TB, IN_W), lambda i: (i, 0)),
            pl.BlockSpec((TOT_W, IN_W), lambda i: (0, 0)),
            pl.BlockSpec((HEAD_W, IN_W), lambda i: (0, 0)),
            pl.BlockSpec((1, TOT_W), lambda i: (0, 0)),
            pl.BlockSpec((1, HEAD_W), lambda i: (0, 0)),
        ],
        out_specs=[
            pl.BlockSpec((TB, TOT_W), lambda i: (i, 0)),
            pl.BlockSpec((TB, SEG), lambda i: (i, 0)),
        ],
        out_shape=[
            jax.ShapeDtypeStruct((B, TOT_W), jnp.bfloat16),
            jax.ShapeDtypeStruct((B, SEG), jnp.float32),
        ],
        compiler_params=pltpu.CompilerParams(
            dimension_semantics=("parallel",)),
    )(x, W1, W1h, b1, bh)


def _make_walk(B):
    b_per_w = B // NW
    mesh = plsc.VectorSubcoreMesh(core_axis_name="c", subcore_axis_name="s")

    @functools.partial(
        pl.kernel, mesh=mesh,
        out_type=jax.ShapeDtypeStruct((B * P,), jnp.float32),
        scratch_types=[
            pltpu.VMEM((b_per_w * SEG,), jnp.float32),
            pltpu.VMEM((b_per_w * P,), jnp.float32),
        ],
        compiler_params=pltpu.CompilerParams(needs_layout_passes=False),
    )
    def walk(dec_hbm, out_hbm, dec_v, out_v):
        wid = lax.axis_index("s") * 2 + lax.axis_index("c")
        base = wid * b_per_w
        pltpu.sync_copy(dec_hbm.at[pl.ds(base * SEG, b_per_w * SEG)], dec_v)
        lanes = lax.iota(jnp.int32, L)
        for batch in range(b_per_w // L):
            rowbase = (batch * L + lanes) * SEG
            for p in range(P):
                mv0 = plsc.load_gather(
                    dec_v, [rowbase + p * GPAD])
                g1 = 1.0 + mv0
                mv1 = plsc.load_gather(
                    dec_v, [rowbase + p * GPAD + g1.astype(jnp.int32)])
                g2 = 1.0 + 4.0 * g1 + mv1
                mv2 = plsc.load_gather(
                    dec_v, [rowbase + p * GPAD + g2.astype(jnp.int32)])
                g3 = 1.0 + 4.0 * g2 + mv2
                plsc.store_scatter(
                    out_v, [(batch * L + lanes) * P + p], g3)
        pltpu.sync_copy(out_v, out_hbm.at[pl.ds(base * P, b_per_w * P)])

    return walk


def _stage3_body(a_ref, g3_ref, s8_ref, lo_ref, hi_ref, w2_ref, o_ref):
    g3e = jax.lax.dot_general(
        g3_ref[...].astype(jnp.bfloat16), s8_ref[...],
        (((1,), (0,)), ((), ())), preferred_element_type=jnp.float32)
    keep = (g3e >= lo_ref[...]) & (g3e < hi_ref[...])
    am = jnp.where(keep, a_ref[...].astype(jnp.float32), 0.0).astype(
        jnp.bfloat16)
    o_ref[...] = jax.lax.dot_general(
        am, w2_ref[...], (((1,), (1,)), ((), ())),
        preferred_element_type=jnp.float32)


@jax.jit
def _stage3(a, g3, S8, lo, hi, W2):
    B = a.shape[0]
    return pl.pallas_call(
        _stage3_body,
        grid=(B // TB,),
        in_specs=[
            pl.BlockSpec((TB, TOT_W), lambda i: (i, 0)),
            pl.BlockSpec((TB, P), lambda i: (i, 0)),
            pl.BlockSpec((P, TOT_W), lambda i: (0, 0)),
            pl.BlockSpec((1, TOT_W), lambda i: (0, 0)),
            pl.BlockSpec((1, TOT_W), lambda i: (0, 0)),
            pl.BlockSpec((OUT_W, TOT_W), lambda i: (0, 0)),
        ],
        out_specs=pl.BlockSpec((TB, OUT_W), lambda i: (i, 0)),
        out_shape=jax.ShapeDtypeStruct((B, OUT_W), jnp.float32),
        compiler_params=pltpu.CompilerParams(
            dimension_semantics=("parallel",)),
    )(a, g3, S8, lo, hi, W2)


def kernel(oldx, W_in, b_in, W_out):
    x = oldx.reshape(-1, IN_W)
    B = x.shape[0]

    W1 = W_in.astype(jnp.bfloat16)
    W2 = W_out.astype(jnp.bfloat16)
    b1 = b_in.reshape(1, TOT_W)

    Wi = W_in.reshape(P, G, FANOUT, IN_W)
    bi = b_in.reshape(P, G, FANOUT)
    Wh = jnp.transpose(Wi[:, :N_HEAD_G], (2, 0, 1, 3))
    Wh = jnp.pad(Wh, ((0, 0), (0, 0), (0, GPAD - N_HEAD_G), (0, 0)))
    W1h = Wh.reshape(HEAD_W, IN_W).astype(jnp.bfloat16)
    bhh = jnp.transpose(bi[:, :N_HEAD_G], (2, 0, 1))
    bhh = jnp.pad(bhh, ((0, 0), (0, 0), (0, GPAD - N_HEAD_G)))
    bh = bhh.reshape(1, HEAD_W)

    itot = jnp.arange(TOT_W)
    S8 = (jnp.arange(P)[:, None] == itot[None, :] // NPG).astype(
        jnp.bfloat16)

    gj = (itot % NPG) // FANOUT
    lo = jnp.where(gj == 0, 21,
                   jnp.where(gj < 5, 21 + 16 * (gj - 1),
                             jnp.where(gj < 21, 21 + 4 * (gj - 5), gj)))
    hi = jnp.where(gj == 0, 85,
                   jnp.where(gj < 5, 21 + 16 * gj,
                             jnp.where(gj < 21, 25 + 4 * (gj - 5), gj + 1)))
    lo = lo.astype(jnp.float32).reshape(1, TOT_W)
    hi = hi.astype(jnp.float32).reshape(1, TOT_W)

    a, dec = _stage1(x, W1, W1h, b1, bh)
    g3 = _make_walk(B)(dec.reshape(B * SEG)).reshape(B, P)
    out = _stage3(a, g3, S8, lo, hi, W2)
    return out.reshape(oldx.shape)


# speedup vs baseline: 1.0220x; 1.0220x over previous
import functools

import jax
import jax.numpy as jnp
from jax import lax
from jax.experimental import pallas as pl
from jax.experimental.pallas import tpu as pltpu
from jax.experimental.pallas import tpu_sc as plsc

IN_W = 2048
OUT_W = 2048
P = 8
FANOUT = 4
G = 85
NPG = G * FANOUT
N_HEAD_G = 21
GPAD = 32
SEG = P * GPAD
HEAD_W = FANOUT * SEG
TOT_W = P * NPG

TB = 1024
NW = 32
L = 16


def _gelu_exact(x):
    return 0.5 * x * (1.0 + jax.lax.erf(x * (2.0 ** -0.5)))


def _gelu_tanh(x):
    c = 0.7978845608028654
    return 0.5 * x * (1.0 + jnp.tanh(c * (x + 0.044715 * x * x * x)))


def _stage1_body(x_ref, w1_ref, w1h_ref, b1_ref, bh_ref, a_ref, dec_ref):
    x_bf = x_ref[...].astype(jnp.bfloat16)
    z = jax.lax.dot_general(
        x_bf, w1_ref[...], (((1,), (1,)), ((), ())),
        preferred_element_type=jnp.float32)
    a_ref[...] = _gelu_tanh(z + b1_ref[...]).astype(jnp.bfloat16)
    zh = jax.lax.dot_general(
        x_bf, w1h_ref[...], (((1,), (1,)), ((), ())),
        preferred_element_type=jnp.float32)
    ah = _gelu_exact(zh + bh_ref[...])
    a0 = ah[:, 0 * SEG:1 * SEG]
    a1 = ah[:, 1 * SEG:2 * SEG]
    a2 = ah[:, 2 * SEG:3 * SEG]
    a3 = ah[:, 3 * SEG:4 * SEG]
    one = jnp.float32(1.0)
    dec = jnp.where(a1 > a0, one, 0.0)
    m = jnp.maximum(a0, a1)
    dec = jnp.where(a2 > m, 2.0, dec)
    m = jnp.maximum(m, a2)
    dec_ref[...] = jnp.where(a3 > m, 3.0, dec)


@jax.jit
def _stage1(x, W1, W1h, b1, bh):
    B = x.shape[0]
    return pl.pallas_call(
        _stage1_body,
        grid=(B // TB,),
        in_specs=[
            pl.BlockSpec((TB, IN_W), lambda i: (i, 0)),
            pl.BlockSpec((TOT_W, IN_W), lambda i: (0, 0)),
            pl.BlockSpec((HEAD_W, IN_W), lambda i: (0, 0)),
            pl.BlockSpec((1, TOT_W), lambda i: (0, 0)),
            pl.BlockSpec((1, HEAD_W), lambda i: (0, 0)),
        ],
        out_specs=[
            pl.BlockSpec((TB, TOT_W), lambda i: (i, 0)),
            pl.BlockSpec((TB, SEG), lambda i: (i, 0)),
        ],
        out_shape=[
            jax.ShapeDtypeStruct((B, TOT_W), jnp.bfloat16),
            jax.ShapeDtypeStruct((B, SEG), jnp.float32),
        ],
        compiler_params=pltpu.CompilerParams(
            dimension_semantics=("parallel",)),
    )(x, W1, W1h, b1, bh)


def _make_walk(B):
    b_per_w = B // NW
    mesh = plsc.VectorSubcoreMesh(core_axis_name="c", subcore_axis_name="s")

    @functools.partial(
        pl.kernel, mesh=mesh,
        out_type=jax.ShapeDtypeStruct((B * P,), jnp.float32),
        scratch_types=[
            pltpu.VMEM((b_per_w * SEG,), jnp.float32),
            pltpu.VMEM((b_per_w * P,), jnp.float32),
        ],
        compiler_params=pltpu.CompilerParams(needs_layout_passes=False),
    )
    def walk(dec_hbm, out_hbm, dec_v, out_v):
        wid = lax.axis_index("s") * 2 + lax.axis_index("c")
        base = wid * b_per_w
        pltpu.sync_copy(dec_hbm.at[pl.ds(base * SEG, b_per_w * SEG)], dec_v)
        lanes = lax.iota(jnp.int32, L)
        for batch in range(b_per_w // L):
            rowbase = (batch * L + lanes) * SEG
            for p in range(P):
                mv0 = plsc.load_gather(
                    dec_v, [rowbase + p * GPAD])
                g1 = 1.0 + mv0
                mv1 = plsc.load_gather(
                    dec_v, [rowbase + p * GPAD + g1.astype(jnp.int32)])
                g2 = 1.0 + 4.0 * g1 + mv1
                mv2 = plsc.load_gather(
                    dec_v, [rowbase + p * GPAD + g2.astype(jnp.int32)])
                g3 = 1.0 + 4.0 * g2 + mv2
                plsc.store_scatter(
                    out_v, [(batch * L + lanes) * P + p], g3)
        pltpu.sync_copy(out_v, out_hbm.at[pl.ds(base * P, b_per_w * P)])

    return walk


def _stage3_body(a_ref, g3_ref, s8_ref, lo_ref, hi_ref, w2_ref, o_ref):
    g3e = jax.lax.dot_general(
        g3_ref[...].astype(jnp.bfloat16), s8_ref[...],
        (((1,), (0,)), ((), ())), preferred_element_type=jnp.float32)
    keep = (g3e >= lo_ref[...]) & (g3e < hi_ref[...])
    am = jnp.where(keep, a_ref[...], jnp.bfloat16(0.0))
    o_ref[...] = jax.lax.dot_general(
        am, w2_ref[...], (((1,), (1,)), ((), ())),
        preferred_element_type=jnp.float32)


@jax.jit
def _stage3(a, g3, S8, lo, hi, W2):
    B = a.shape[0]
    return pl.pallas_call(
        _stage3_body,
        grid=(B // TB,),
        in_specs=[
            pl.BlockSpec((TB, TOT_W), lambda i: (i, 0)),
            pl.BlockSpec((TB, P), lambda i: (i, 0)),
            pl.BlockSpec((P, TOT_W), lambda i: (0, 0)),
            pl.BlockSpec((1, TOT_W), lambda i: (0, 0)),
            pl.BlockSpec((1, TOT_W), lambda i: (0, 0)),
            pl.BlockSpec((OUT_W, TOT_W), lambda i: (0, 0)),
        ],
        out_specs=pl.BlockSpec((TB, OUT_W), lambda i: (i, 0)),
        out_shape=jax.ShapeDtypeStruct((B, OUT_W), jnp.float32),
        compiler_params=pltpu.CompilerParams(
            dimension_semantics=("parallel",)),
    )(a, g3, S8, lo, hi, W2)


def kernel(oldx, W_in, b_in, W_out):
    x = oldx.reshape(-1, IN_W)
    B = x.shape[0]

    W1 = W_in.astype(jnp.bfloat16)
    W2 = W_out.astype(jnp.bfloat16)
    b1 = b_in.reshape(1, TOT_W)

    Wi = W_in.reshape(P, G, FANOUT, IN_W)
    bi = b_in.reshape(P, G, FANOUT)
    Wh = jnp.transpose(Wi[:, :N_HEAD_G], (2, 0, 1, 3))
    Wh = jnp.pad(Wh, ((0, 0), (0, 0), (0, GPAD - N_HEAD_G), (0, 0)))
    W1h = Wh.reshape(HEAD_W, IN_W).astype(jnp.bfloat16)
    bhh = jnp.transpose(bi[:, :N_HEAD_G], (2, 0, 1))
    bhh = jnp.pad(bhh, ((0, 0), (0, 0), (0, GPAD - N_HEAD_G)))
    bh = bhh.reshape(1, HEAD_W)

    itot = jnp.arange(TOT_W)
    S8 = (jnp.arange(P)[:, None] == itot[None, :] // NPG).astype(
        jnp.bfloat16)

    gj = (itot % NPG) // FANOUT
    lo = jnp.where(gj == 0, 21,
                   jnp.where(gj < 5, 21 + 16 * (gj - 1),
                             jnp.where(gj < 21, 21 + 4 * (gj - 5), gj)))
    hi = jnp.where(gj == 0, 85,
                   jnp.where(gj < 5, 21 + 16 * gj,
                             jnp.where(gj < 21, 25 + 4 * (gj - 5), gj + 1)))
    lo = lo.astype(jnp.float32).reshape(1, TOT_W)
    hi = hi.astype(jnp.float32).reshape(1, TOT_W)

    a, dec = _stage1(x, W1, W1h, b1, bh)
    g3 = _make_walk(B)(dec.reshape(B * SEG)).reshape(B, P)
    out = _stage3(a, g3, S8, lo, hi, W2)
    return out.reshape(oldx.shape)


# speedup vs baseline: 1.0275x; 1.0054x over previous
import functools

import jax
import jax.numpy as jnp
from jax import lax
from jax.experimental import pallas as pl
from jax.experimental.pallas import tpu as pltpu
from jax.experimental.pallas import tpu_sc as plsc

IN_W = 2048
OUT_W = 2048
P = 8
FANOUT = 4
G = 85
NPG = G * FANOUT
N_HEAD_G = 21
GPAD = 32
SEG = P * GPAD
HEAD_W = FANOUT * SEG
TOT_W = P * NPG

TB = 1024
NW = 32
L = 16


def _gelu_exact(x):
    return 0.5 * x * (1.0 + jax.lax.erf(x * (2.0 ** -0.5)))


def _gelu_tanh(x):
    c = 0.7978845608028654
    return 0.5 * x * (1.0 + jnp.tanh(c * (x + 0.044715 * x * x * x)))


def _stage1_body(x_ref, w1_ref, w1h_ref, b1_ref, bh_ref, a_ref, dec_ref):
    x_bf = x_ref[...].astype(jnp.bfloat16)
    z = jax.lax.dot_general(
        x_bf, w1_ref[...], (((1,), (1,)), ((), ())),
        preferred_element_type=jnp.float32)
    a_ref[...] = _gelu_tanh(z + b1_ref[...]).astype(jnp.bfloat16)
    zh = jax.lax.dot_general(
        x_bf, w1h_ref[...], (((1,), (1,)), ((), ())),
        preferred_element_type=jnp.float32)
    ah = _gelu_exact(zh + bh_ref[...])
    a0 = ah[:, 0 * SEG:1 * SEG]
    a1 = ah[:, 1 * SEG:2 * SEG]
    a2 = ah[:, 2 * SEG:3 * SEG]
    a3 = ah[:, 3 * SEG:4 * SEG]
    one = jnp.float32(1.0)
    dec = jnp.where(a1 > a0, one, 0.0)
    m = jnp.maximum(a0, a1)
    dec = jnp.where(a2 > m, 2.0, dec)
    m = jnp.maximum(m, a2)
    dec_ref[...] = jnp.where(a3 > m, 3.0, dec)


@jax.jit
def _stage1(x, W1, W1h, b1, bh):
    B = x.shape[0]
    return pl.pallas_call(
        _stage1_body,
        grid=(B // TB,),
        in_specs=[
            pl.BlockSpec((TB, IN_W), lambda i: (i, 0)),
            pl.BlockSpec((TOT_W, IN_W), lambda i: (0, 0)),
            pl.BlockSpec((HEAD_W, IN_W), lambda i: (0, 0)),
            pl.BlockSpec((1, TOT_W), lambda i: (0, 0)),
            pl.BlockSpec((1, HEAD_W), lambda i: (0, 0)),
        ],
        out_specs=[
            pl.BlockSpec((TB, TOT_W), lambda i: (i, 0)),
            pl.BlockSpec((TB, SEG), lambda i: (i, 0)),
        ],
        out_shape=[
            jax.ShapeDtypeStruct((B, TOT_W), jnp.bfloat16),
            jax.ShapeDtypeStruct((B, SEG), jnp.float32),
        ],
        compiler_params=pltpu.CompilerParams(
            dimension_semantics=("parallel",)),
    )(x, W1, W1h, b1, bh)


def _make_walk(B):
    b_per_w = B // NW
    mesh = plsc.VectorSubcoreMesh(core_axis_name="c", subcore_axis_name="s")

    @functools.partial(
        pl.kernel, mesh=mesh,
        out_type=jax.ShapeDtypeStruct((B * P,), jnp.float32),
        scratch_types=[
            pltpu.VMEM((b_per_w * SEG,), jnp.float32),
            pltpu.VMEM((b_per_w * P,), jnp.float32),
        ],
        compiler_params=pltpu.CompilerParams(needs_layout_passes=False),
    )
    def walk(dec_hbm, out_hbm, dec_v, out_v):
        wid = lax.axis_index("s") * 2 + lax.axis_index("c")
        base = wid * b_per_w
        pltpu.sync_copy(dec_hbm.at[pl.ds(base * SEG, b_per_w * SEG)], dec_v)
        lanes = lax.iota(jnp.int32, L)
        for batch in range(b_per_w // L):
            rowbase = (batch * L + lanes) * SEG
            for p in range(P):
                mv0 = plsc.load_gather(
                    dec_v, [rowbase + p * GPAD])
                g1 = 1.0 + mv0
                mv1 = plsc.load_gather(
                    dec_v, [rowbase + p * GPAD + g1.astype(jnp.int32)])
                g2 = 1.0 + 4.0 * g1 + mv1
                mv2 = plsc.load_gather(
                    dec_v, [rowbase + p * GPAD + g2.astype(jnp.int32)])
                g3 = 1.0 + 4.0 * g2 + mv2
                plsc.store_scatter(
                    out_v, [(batch * L + lanes) * P + p], g3)
        pltpu.sync_copy(out_v, out_hbm.at[pl.ds(base * P, b_per_w * P)])

    return walk


def _stage3_body(a_ref, g3_ref, s8_ref, lo_ref, hi_ref, w2_ref, o_ref):
    g3e = jax.lax.dot_general(
        g3_ref[...].astype(jnp.bfloat16), s8_ref[...],
        (((1,), (0,)), ((), ())), preferred_element_type=jnp.float32)
    keep = (g3e >= lo_ref[...]) & (g3e < hi_ref[...])
    am = jnp.where(keep, a_ref[...].astype(jnp.float32), 0.0).astype(
        jnp.bfloat16)
    o_ref[...] = jax.lax.dot_general(
        am, w2_ref[...], (((1,), (1,)), ((), ())),
        preferred_element_type=jnp.float32)


@jax.jit
def _stage3(a, g3, S8, lo, hi, W2):
    B = a.shape[0]
    return pl.pallas_call(
        _stage3_body,
        grid=(B // TB,),
        in_specs=[
            pl.BlockSpec((TB, TOT_W), lambda i: (i, 0)),
            pl.BlockSpec((TB, P), lambda i: (i, 0)),
            pl.BlockSpec((P, TOT_W), lambda i: (0, 0)),
            pl.BlockSpec((1, TOT_W), lambda i: (0, 0)),
            pl.BlockSpec((1, TOT_W), lambda i: (0, 0)),
            pl.BlockSpec((OUT_W, TOT_W), lambda i: (0, 0)),
        ],
        out_specs=pl.BlockSpec((TB, OUT_W), lambda i: (i, 0)),
        out_shape=jax.ShapeDtypeStruct((B, OUT_W), jnp.float32),
        compiler_params=pltpu.CompilerParams(
            dimension_semantics=("parallel",)),
    )(a, g3, S8, lo, hi, W2)


def kernel(oldx, W_in, b_in, W_out):
    x = oldx.reshape(-1, IN_W)
    B = x.shape[0]

    W1 = W_in.astype(jnp.bfloat16)
    W2 = W_out.astype(jnp.bfloat16)
    b1 = b_in.reshape(1, TOT_W)

    Wi = W_in.reshape(P, G, FANOUT, IN_W)
    bi = b_in.reshape(P, G, FANOUT)
    Wh = jnp.transpose(Wi[:, :N_HEAD_G], (2, 0, 1, 3))
    Wh = jnp.pad(Wh, ((0, 0), (0, 0), (0, GPAD - N_HEAD_G), (0, 0)))
    W1h = Wh.reshape(HEAD_W, IN_W).astype(jnp.bfloat16)
    bhh = jnp.transpose(bi[:, :N_HEAD_G], (2, 0, 1))
    bhh = jnp.pad(bhh, ((0, 0), (0, 0), (0, GPAD - N_HEAD_G)))
    bh = bhh.reshape(1, HEAD_W)

    itot = jnp.arange(TOT_W)
    S8 = (jnp.arange(P)[:, None] == itot[None, :] // NPG).astype(
        jnp.bfloat16)

    gj = (itot % NPG) // FANOUT
    lo = jnp.where(gj == 0, 21,
                   jnp.where(gj < 5, 21 + 16 * (gj - 1),
                             jnp.where(gj < 21, 21 + 4 * (gj - 5), gj)))
    hi = jnp.where(gj == 0, 85,
                   jnp.where(gj < 5, 21 + 16 * gj,
                             jnp.where(gj < 21, 25 + 4 * (gj - 5), gj + 1)))
    lo = lo.astype(jnp.float32).reshape(1, TOT_W)
    hi = hi.astype(jnp.float32).reshape(1, TOT_W)

    a, dec = _stage1(x, W1, W1h, b1, bh)
    g3 = _make_walk(B)(dec.reshape(B * SEG)).reshape(B, P)
    out = _stage3(a, g3, S8, lo, hi, W2)
    return out.reshape(oldx.shape)
